# baseline (speedup 1.0000x reference)
import jax
import jax.numpy as jnp
from jax import lax
from jax.experimental import pallas as pl
from jax.experimental.pallas import tpu as pltpu
from jax.experimental.pallas import tpu_sc as plsc

NUM_FIELDS = 26
VOCAB = 100000
EMB = 32
BATCH = 16384
TOTAL_DIM = 39
RES = TOTAL_DIM - NUM_FIELDS
OUT_DIM = EMB + RES

NC = 2
NS = 16
NW = NC * NS
L = 16

QB = 4096
NQ = BATCH // QB
UNROLL = 8
SEG0 = 50048
SEG1 = VOCAB - SEG0


def _emb_body(vft_hbm, tbl_hbm, out_hbm, segA_v, segB_v, acc_v, idx0_v,
              idx1_v, semA, semB, sem2):
    e = lax.axis_index("s") * NC + lax.axis_index("c")
    idx_bufs = (idx0_v, idx1_v)

    zeros = jnp.zeros((L,), jnp.float32)

    def zero_acc(i, _):
        b = i * L * UNROLL
        for u in range(UNROLL):
            acc_v[pl.ds(b + u * L, L)] = zeros
        return _

    lax.fori_loop(0, BATCH // (L * UNROLL), zero_acc, 0)

    def segA_copy(f):
        return pltpu.make_async_copy(
            tbl_hbm.at[f, e, pl.ds(0, SEG0)], segA_v, semA
        )

    def segB_copy(f):
        return pltpu.make_async_copy(
            tbl_hbm.at[f, e, pl.ds(SEG0, SEG1)], segB_v, semB
        )

    def gather_pass(f, seg_v, lo):
        for q in range(NQ):
            buf = idx_bufs[q % 2]
            if q + 1 < NQ:
                nbuf = idx_bufs[(q + 1) % 2]
                pltpu.async_copy(
                    vft_hbm.at[f, pl.ds((q + 1) * QB, QB)], nbuf, sem2
                )

            def gath(i, _, buf=buf, q=q):
                b = i * L * UNROLL
                for u in range(UNROLL):
                    ix = buf[pl.ds(b + u * L, L)].astype(jnp.int32)
                    if lo == 0:
                        mask = ix < SEG0
                        ixs = ix
                    else:
                        mask = ix >= SEG0
                        ixs = ix - SEG0
                    vals = plsc.load_gather(seg_v, [ixs], mask=mask)
                    vals = jnp.where(mask, vals, 0.0)
                    o = q * QB + b + u * L
                    acc_v[pl.ds(o, L)] = acc_v[pl.ds(o, L)] + vals
                return _

            lax.fori_loop(0, QB // (L * UNROLL), gath, 0)
            if q + 1 < NQ:
                pltpu.make_async_copy(
                    vft_hbm.at[f, pl.ds((q + 1) * QB, QB)], nbuf, sem2
                ).wait()

    segA_copy(0).start()

    def do_field(f, _):
        segB_copy(f).start()
        pltpu.async_copy(vft_hbm.at[f, pl.ds(0, QB)], idx0_v, sem2)
        segA_copy(f).wait()
        pltpu.make_async_copy(
            vft_hbm.at[f, pl.ds(0, QB)], idx0_v, sem2
        ).wait()
        gather_pass(f, segA_v, 0)
        segB_copy(f).wait()

        @pl.when(f + 1 < NUM_FIELDS)
        def _next_low():
            segA_copy(f + 1).start()

        pltpu.sync_copy(vft_hbm.at[f, pl.ds(0, QB)], idx0_v)
        gather_pass(f, segB_v, SEG0)
        return _

    lax.fori_loop(0, NUM_FIELDS, do_field, 0)

    pltpu.sync_copy(acc_v, out_hbm.at[e])

    @pl.when(e < RES)
    def _():
        pltpu.sync_copy(vft_hbm.at[NUM_FIELDS + e], acc_v)
        pltpu.sync_copy(acc_v, out_hbm.at[EMB + e])


@jax.jit
def _emb_kernel(vft, tbl_t):
    mesh = plsc.VectorSubcoreMesh(
        core_axis_name="c", subcore_axis_name="s", num_cores=NC, num_subcores=NS
    )
    out_t = pl.kernel(
        _emb_body,
        out_type=jax.ShapeDtypeStruct((OUT_DIM, BATCH), jnp.float32),
        mesh=mesh,
        compiler_params=pltpu.CompilerParams(
            needs_layout_passes=False, use_tc_tiling_on_sc=True
        ),
        scratch_types=[
            pltpu.VMEM((SEG0,), jnp.float32),
            pltpu.VMEM((SEG1,), jnp.float32),
            pltpu.VMEM((BATCH,), jnp.float32),
            pltpu.VMEM((QB,), jnp.float32),
            pltpu.VMEM((QB,), jnp.float32),
            pltpu.SemaphoreType.DMA,
            pltpu.SemaphoreType.DMA,
            pltpu.SemaphoreType.DMA,
        ],
    )(vft, tbl_t)
    return out_t.T


def kernel(v_f, emb_tables):
    return _emb_kernel(v_f.T, emb_tables.transpose(0, 2, 1))


# speedup vs baseline: 1.4277x; 1.4277x over previous
import jax
import jax.numpy as jnp
from jax import lax
from jax.experimental import pallas as pl
from jax.experimental.pallas import tpu as pltpu
from jax.experimental.pallas import tpu_sc as plsc

NUM_FIELDS = 26
VOCAB = 100000
EMB = 32
BATCH = 16384
TOTAL_DIM = 39
RES = TOTAL_DIM - NUM_FIELDS
OUT_DIM = EMB + RES

NC = 2
NS = 16
NW = NC * NS
L = 16

QB = 4096
NQ = BATCH // QB
UNROLL = 16


def _emb_body(vft_hbm, tbl_hbm, out_hbm, vocab_v, acc_v, idx0_v, idx1_v,
              sem, sem2):
    e = lax.axis_index("s") * NC + lax.axis_index("c")
    idx_bufs = (idx0_v, idx1_v)

    def field_quarters(f, first):
        for q in range(NQ):
            buf = idx_bufs[q % 2]
            if q + 1 < NQ:
                nbuf = idx_bufs[(q + 1) % 2]
                pltpu.async_copy(
                    vft_hbm.at[f, pl.ds((q + 1) * QB, QB)], nbuf, sem2
                )

            def gath(i, _, buf=buf, q=q):
                b = i * L * UNROLL
                for u in range(UNROLL):
                    ix = buf[pl.ds(b + u * L, L)].astype(jnp.int32)
                    vals = plsc.load_gather(vocab_v, [ix])
                    o = q * QB + b + u * L
                    if first:
                        acc_v[pl.ds(o, L)] = vals
                    else:
                        acc_v[pl.ds(o, L)] = acc_v[pl.ds(o, L)] + vals
                return _

            lax.fori_loop(0, QB // (L * UNROLL), gath, 0)
            if q + 1 < NQ:
                pltpu.make_async_copy(
                    vft_hbm.at[f, pl.ds((q + 1) * QB, QB)], nbuf, sem2
                ).wait()

    def stage_field(f):
        pltpu.async_copy(tbl_hbm.at[f, e], vocab_v, sem)
        pltpu.async_copy(vft_hbm.at[f, pl.ds(0, QB)], idx0_v, sem2)
        pltpu.make_async_copy(tbl_hbm.at[f, e], vocab_v, sem).wait()
        pltpu.make_async_copy(
            vft_hbm.at[f, pl.ds(0, QB)], idx0_v, sem2
        ).wait()

    stage_field(0)
    field_quarters(0, True)

    def do_field(f, _):
        stage_field(f)
        field_quarters(f, False)
        return _

    lax.fori_loop(1, NUM_FIELDS, do_field, 0)

    pltpu.sync_copy(acc_v, out_hbm.at[e])

    @pl.when(e < RES)
    def _():
        pltpu.sync_copy(vft_hbm.at[NUM_FIELDS + e], acc_v)
        pltpu.sync_copy(acc_v, out_hbm.at[EMB + e])


@jax.jit
def _emb_kernel(vft, tbl_t):
    mesh = plsc.VectorSubcoreMesh(
        core_axis_name="c", subcore_axis_name="s", num_cores=NC, num_subcores=NS
    )
    out_t = pl.kernel(
        _emb_body,
        out_type=jax.ShapeDtypeStruct((OUT_DIM, BATCH), jnp.float32),
        mesh=mesh,
        compiler_params=pltpu.CompilerParams(
            needs_layout_passes=False, use_tc_tiling_on_sc=True
        ),
        scratch_types=[
            pltpu.VMEM((VOCAB,), jnp.float32),
            pltpu.VMEM((BATCH,), jnp.float32),
            pltpu.VMEM((QB,), jnp.float32),
            pltpu.VMEM((QB,), jnp.float32),
            pltpu.SemaphoreType.DMA,
            pltpu.SemaphoreType.DMA,
        ],
    )(vft, tbl_t)
    return out_t.T


def kernel(v_f, emb_tables):
    return _emb_kernel(v_f.T, emb_tables.transpose(0, 2, 1))


# speedup vs baseline: 1.9042x; 1.3338x over previous
import jax
import jax.numpy as jnp
from jax import lax
from jax.experimental import pallas as pl
from jax.experimental.pallas import tpu as pltpu
from jax.experimental.pallas import tpu_sc as plsc

NUM_FIELDS = 26
VOCAB = 100000
EMB = 32
BATCH = 16384
TOTAL_DIM = 39
RES = TOTAL_DIM - NUM_FIELDS
OUT_DIM = EMB + RES

NC = 2
NS = 16
NW = NC * NS
L = 16

QB = 4096
NQ = BATCH // QB
UNROLL = 8


def _emb_body(vft_hbm, tbl_hbm, out_hbm, vocab_v, acc_v, idx0_v, idx1_v,
              sem, sem2):
    e = lax.axis_index("s") * NC + lax.axis_index("c")
    idx_bufs = (idx0_v, idx1_v)

    def field_quarters(f, first):
        for q in range(NQ):
            buf = idx_bufs[q % 2]
            if q + 1 < NQ:
                nbuf = idx_bufs[(q + 1) % 2]
                pltpu.async_copy(
                    vft_hbm.at[f, pl.ds((q + 1) * QB, QB)], nbuf, sem2
                )

            @plsc.parallel_loop(0, QB // L, unroll=UNROLL)
            def gath(i, buf=buf, q=q):
                b = i * L
                ix = buf[pl.ds(b, L)].astype(jnp.int32)
                vals = plsc.load_gather(vocab_v, [ix])
                o = q * QB + b
                if first:
                    acc_v[pl.ds(o, L)] = vals
                else:
                    acc_v[pl.ds(o, L)] = acc_v[pl.ds(o, L)] + vals
            if q + 1 < NQ:
                pltpu.make_async_copy(
                    vft_hbm.at[f, pl.ds((q + 1) * QB, QB)], nbuf, sem2
                ).wait()

    def stage_field(f):
        pltpu.async_copy(tbl_hbm.at[f, e], vocab_v, sem)
        pltpu.async_copy(vft_hbm.at[f, pl.ds(0, QB)], idx0_v, sem2)
        pltpu.make_async_copy(tbl_hbm.at[f, e], vocab_v, sem).wait()
        pltpu.make_async_copy(
            vft_hbm.at[f, pl.ds(0, QB)], idx0_v, sem2
        ).wait()

    stage_field(0)
    field_quarters(0, True)

    def do_field(f, _):
        stage_field(f)
        field_quarters(f, False)
        return _

    lax.fori_loop(1, NUM_FIELDS, do_field, 0)

    pltpu.sync_copy(acc_v, out_hbm.at[e])

    @pl.when(e < RES)
    def _():
        pltpu.sync_copy(vft_hbm.at[NUM_FIELDS + e], acc_v)
        pltpu.sync_copy(acc_v, out_hbm.at[EMB + e])


@jax.jit
def _emb_kernel(vft, tbl_t):
    mesh = plsc.VectorSubcoreMesh(
        core_axis_name="c", subcore_axis_name="s", num_cores=NC, num_subcores=NS
    )
    out_t = pl.kernel(
        _emb_body,
        out_type=jax.ShapeDtypeStruct((OUT_DIM, BATCH), jnp.float32),
        mesh=mesh,
        compiler_params=pltpu.CompilerParams(
            needs_layout_passes=False, use_tc_tiling_on_sc=True
        ),
        scratch_types=[
            pltpu.VMEM((VOCAB,), jnp.float32),
            pltpu.VMEM((BATCH,), jnp.float32),
            pltpu.VMEM((QB,), jnp.float32),
            pltpu.VMEM((QB,), jnp.float32),
            pltpu.SemaphoreType.DMA,
            pltpu.SemaphoreType.DMA,
        ],
    )(vft, tbl_t)
    return out_t.T


def kernel(v_f, emb_tables):
    return _emb_kernel(v_f.T, emb_tables.transpose(0, 2, 1))
